# Initial kernel scaffold; baseline (speedup 1.0000x reference)
import jax
import jax.numpy as jnp
from jax.experimental import pallas as pl


def kernel(edge_index, user_index, item_index, user_emb, item_emb):
    raise NotImplementedError("write your pallas kernel here")


# speedup vs baseline: 6.4504x; 6.4504x over previous
import functools

import jax
import jax.numpy as jnp
from jax import lax
from jax.experimental import pallas as pl
from jax.experimental.pallas import tpu as pltpu
from jax.experimental.pallas import tpu_sc as plsc

N_REAL = 10000
NPAD = 10240
D = 128
NT = 32
CH = 128
NCH = 80
GRP = 4
NG = NCH // GRP
EPT = NCH * CH
ROWS_PER_TILE = NPAD // 16

_mesh = plsc.VectorSubcoreMesh(core_axis_name="c", subcore_axis_name="s")
_sc_params = pltpu.CompilerParams(needs_layout_passes=False)


@functools.partial(
    pl.kernel,
    out_type=jax.ShapeDtypeStruct((NT, NPAD), jnp.float32),
    mesh=_mesh,
    compiler_params=_sc_params,
    scratch_types=[
        pltpu.VMEM((NG, GRP, 2, CH), jnp.int32),
        pltpu.VMEM((NPAD,), jnp.float32),
    ],
)
def _deg_kernel(idx_hbm, deg_hbm, idx_v, deg_v):
    cid = lax.axis_index("c")
    sid = lax.axis_index("s")
    wid = cid * 16 + sid
    pltpu.sync_copy(idx_hbm.at[wid], idx_v)

    zv = jnp.zeros((16,), jnp.float32)

    def zbody(i, _):
        deg_v[pl.ds(i * 16, 16)] = zv
        return _

    lax.fori_loop(0, NPAD // 16, zbody, None)

    ones = jnp.ones((16,), jnp.float32)

    def gbody(g, _):
        for k in range(GRP):
            def hbody(h, __):
                idx = idx_v[g, k, 1, pl.ds(h * 16, 16)]
                plsc.addupdate_scatter(deg_v, [idx], ones)
                return __
            lax.fori_loop(0, CH // 16, hbody, None)
        return _

    lax.fori_loop(0, NG, gbody, None)
    pltpu.sync_copy(deg_v, deg_hbm.at[wid])


@functools.partial(
    pl.kernel,
    out_type=jax.ShapeDtypeStruct((2 * NPAD, D), jnp.float32),
    mesh=_mesh,
    compiler_params=_sc_params,
    scratch_types=[
        pltpu.VMEM((2, GRP, 2, CH), jnp.int32),
        pltpu.VMEM((CH, D), jnp.float32),
        pltpu.VMEM((CH, D), jnp.float32),
        pltpu.MemorySpace.VMEM_SHARED((NPAD, D), jnp.float32),
        pltpu.SemaphoreType.DMA,
        pltpu.SemaphoreType.DMA,
        pltpu.SemaphoreType.DMA,
    ],
)
def _prop_kernel(y_hbm, idx_hbm, u_hbm, ig, bufa, bufb, acc_sh,
                 sema, semb, semi):
    cid = lax.axis_index("c")
    sid = lax.axis_index("s")
    wid = cid * 16 + sid
    bufs = (bufa, bufb)
    sems = (sema, semb)

    zv = jnp.zeros((16,), jnp.float32)

    def zrow(i, _):
        for k in range(D // 16):
            bufa[i, pl.ds(k * 16, 16)] = zv
        return _

    lax.fori_loop(0, CH, zrow, None)
    r0 = sid * ROWS_PER_TILE
    for t in range(ROWS_PER_TILE // CH):
        pltpu.sync_copy(bufa, acc_sh.at[pl.ds(r0 + t * CH, CH)])
    plsc.subcore_barrier()

    pltpu.sync_copy(idx_hbm.at[wid, 0], ig.at[0])
    pltpu.async_copy(y_hbm.at[ig.at[0, 0, 0]], bufa, sema)
    pltpu.async_copy(idx_hbm.at[wid, 1], ig.at[1], semi)

    def gbody(g, _):
        gpar = lax.rem(g, 2)
        npar = 1 - gpar
        pltpu.make_async_copy(idx_hbm.at[wid, 0], ig.at[0], semi).wait()
        for k in range(GRP):
            pltpu.make_async_copy(
                y_hbm.at[ig.at[0, 0, 0]], bufs[k % 2], sems[k % 2]).wait()
            if k < GRP - 1:
                nidx = ig.at[gpar, k + 1, 0]
            else:
                nidx = ig.at[npar, 0, 0]
            pltpu.async_copy(
                y_hbm.at[nidx], bufs[(k + 1) % 2], sems[(k + 1) % 2])
            pltpu.sync_copy(bufs[k % 2], acc_sh.at[ig.at[gpar, k, 1]],
                            add=True)
        nxt = jnp.minimum(g + 2, NG - 1)
        pltpu.async_copy(idx_hbm.at[wid, nxt], ig.at[gpar], semi)
        return _

    lax.fori_loop(0, NG, gbody, None)
    pltpu.make_async_copy(idx_hbm.at[wid, 0], ig.at[0], semi).wait()
    pltpu.make_async_copy(y_hbm.at[ig.at[0, 0, 0]], bufa, sema).wait()
    plsc.subcore_barrier()

    for t in range(ROWS_PER_TILE // CH):
        pltpu.sync_copy(acc_sh.at[pl.ds(r0 + t * CH, CH)], bufa)
        pltpu.sync_copy(bufa, u_hbm.at[pl.ds(cid * NPAD + r0 + t * CH, CH)])


@functools.partial(
    pl.kernel,
    out_type=jax.ShapeDtypeStruct((8192, D), jnp.float32),
    mesh=_mesh,
    compiler_params=_sc_params,
    scratch_types=[
        pltpu.VMEM((2, CH), jnp.int32),
        pltpu.VMEM((CH, D), jnp.float32),
        pltpu.SemaphoreType.DMA,
    ],
)
def _gather_kernel(tab_hbm, idx_hbm, o_hbm, idx_v, buf, sem):
    cid = lax.axis_index("c")
    sid = lax.axis_index("s")
    wid = cid * 16 + sid
    pltpu.sync_copy(idx_hbm.at[wid], idx_v)
    for j in range(2):
        pltpu.async_copy(tab_hbm.at[idx_v.at[j]], buf, sem).wait()
        pltpu.sync_copy(buf, o_hbm.at[pl.ds(wid * 2 * CH + j * CH, CH)])


def _prep_body(degp_ref, x0_ref, y_ref, s_ref, s2_ref):
    deg = jnp.sum(degp_ref[...], axis=0)
    pos = deg > 0
    safe = jnp.where(pos, deg, 1.0)
    s = jnp.where(pos, lax.rsqrt(safe), 0.0)
    s2 = jnp.where(pos, 1.0 / safe, 0.0)
    s_ref[...] = s
    s2_ref[...] = s2
    y_ref[...] = x0_ref[...] * s[:, None]


def _comb_body(u_ref, s2_ref, us_ref, y_ref, usum_ref):
    t = u_ref[:NPAD, :] + u_ref[NPAD:, :]
    usum_ref[...] = us_ref[...] + t
    y_ref[...] = t * s2_ref[...][:, None]


def _final_body(u_ref, s_ref, us_ref, x0_ref, out_ref):
    t = u_ref[:NPAD, :] + u_ref[NPAD:, :]
    usum = us_ref[...] + t
    out_ref[...] = x0_ref[...] + usum * s_ref[...][:, None]


def _dot_body(o_ref, score_ref):
    ou = o_ref[:4096, :]
    oi = o_ref[4096:, :]
    score_ref[...] = jnp.sum(ou * oi, axis=1) * (1.0 / 16.0)


_prep_call = pl.pallas_call(
    _prep_body,
    out_shape=(
        jax.ShapeDtypeStruct((NPAD, D), jnp.float32),
        jax.ShapeDtypeStruct((NPAD,), jnp.float32),
        jax.ShapeDtypeStruct((NPAD,), jnp.float32),
    ),
)

_comb_call = pl.pallas_call(
    _comb_body,
    out_shape=(
        jax.ShapeDtypeStruct((NPAD, D), jnp.float32),
        jax.ShapeDtypeStruct((NPAD, D), jnp.float32),
    ),
)

_final_call = pl.pallas_call(
    _final_body,
    out_shape=jax.ShapeDtypeStruct((NPAD, D), jnp.float32),
)

_dot_call = pl.pallas_call(
    _dot_body,
    out_shape=jax.ShapeDtypeStruct((4096,), jnp.float32),
)


def kernel(edge_index, user_index, item_index, user_emb, item_emb):
    x0 = jnp.concatenate([user_emb, item_emb], axis=0)
    x0p = jnp.pad(x0, ((0, NPAD - N_REAL), (0, 0)))
    src = edge_index[0].astype(jnp.int32)
    dst = edge_index[1].astype(jnp.int32)
    padlen = NT * EPT - src.shape[0]
    pad = jnp.full((padlen,), N_REAL, jnp.int32)
    srcp = jnp.concatenate([src, pad]).reshape(NT, NCH, CH)
    dstp = jnp.concatenate([dst, pad]).reshape(NT, NCH, CH)
    idx5 = jnp.stack([srcp, dstp], axis=2).reshape(NT, NG, GRP, 2, CH)
    bidx = jnp.concatenate(
        [user_index.astype(jnp.int32), item_index.astype(jnp.int32)]
    ).reshape(NT, 2, CH)

    deg_parts = _deg_kernel(idx5)
    y, s, s2 = _prep_call(deg_parts, x0p)
    usum = jnp.zeros((NPAD, D), jnp.float32)
    out_full = None
    for layer in range(3):
        u = _prop_kernel(y, idx5)
        if layer < 2:
            y, usum = _comb_call(u, s2, usum)
        else:
            out_full = _final_call(u, s, usum, x0p)

    o = _gather_kernel(out_full, bidx)
    return _dot_call(o)


# speedup vs baseline: 19.0354x; 2.9510x over previous
import functools

import jax
import jax.numpy as jnp
from jax import lax
from jax.experimental import pallas as pl
from jax.experimental.pallas import tpu as pltpu
from jax.experimental.pallas import tpu_sc as plsc

N_REAL = 10000
NPAD = 10240
D = 128
NT = 32
CH = 128
NCH = 80
GRP = 4
NG = NCH // GRP
EPT = NCH * CH
ROWS_PER_TILE = NPAD // 16

_mesh = plsc.VectorSubcoreMesh(core_axis_name="c", subcore_axis_name="s")
_sc_params = pltpu.CompilerParams(needs_layout_passes=False)


@functools.partial(
    pl.kernel,
    out_type=jax.ShapeDtypeStruct((NT, NPAD), jnp.float32),
    mesh=_mesh,
    compiler_params=_sc_params,
    scratch_types=[
        pltpu.VMEM((NG, GRP, 2, CH), jnp.int32),
        pltpu.VMEM((NPAD,), jnp.float32),
    ],
)
def _deg_kernel(idx_hbm, deg_hbm, idx_v, deg_v):
    cid = lax.axis_index("c")
    sid = lax.axis_index("s")
    wid = cid * 16 + sid
    pltpu.sync_copy(idx_hbm.at[wid], idx_v)

    zv = jnp.zeros((16,), jnp.float32)

    def zbody(i, _):
        deg_v[pl.ds(i * 16, 16)] = zv
        return _

    lax.fori_loop(0, NPAD // 16, zbody, None)

    ones = jnp.ones((16,), jnp.float32)

    def gbody(g, _):
        for k in range(GRP):
            def hbody(h, __):
                idx = idx_v[g, k, 1, pl.ds(h * 16, 16)]
                plsc.addupdate_scatter(deg_v, [idx], ones)
                return __
            lax.fori_loop(0, CH // 16, hbody, None)
        return _

    lax.fori_loop(0, NG, gbody, None)
    pltpu.sync_copy(deg_v, deg_hbm.at[wid])


@functools.partial(
    pl.kernel,
    out_type=jax.ShapeDtypeStruct((2 * NPAD, D), jnp.float32),
    mesh=_mesh,
    compiler_params=_sc_params,
    scratch_types=[
        pltpu.VMEM((2, GRP, 2, CH), jnp.int32),
        pltpu.VMEM((CH, D), jnp.float32),
        pltpu.VMEM((CH, D), jnp.float32),
        pltpu.MemorySpace.VMEM_SHARED((NPAD, D), jnp.float32),
        pltpu.SemaphoreType.DMA,
        pltpu.SemaphoreType.DMA,
        pltpu.SemaphoreType.DMA,
        pltpu.SemaphoreType.DMA,
        pltpu.SemaphoreType.DMA,
    ],
)
def _prop_kernel(y_hbm, idx_hbm, u_hbm, ig, bufa, bufb, acc_sh,
                 sema, semb, semsa, semsb, semi):
    cid = lax.axis_index("c")
    sid = lax.axis_index("s")
    wid = cid * 16 + sid
    bufs = (bufa, bufb)
    gsems = (sema, semb)
    ssems = (semsa, semsb)
    BUF_BYTES = CH * D * 4

    pltpu.sync_copy(idx_hbm.at[wid, 0], ig.at[0])
    pltpu.async_copy(idx_hbm.at[wid, 1], ig.at[1], semi)

    zv = jnp.zeros((16,), jnp.float32)

    def zrow(i, _):
        for k in range(D // 16):
            bufa[i, pl.ds(k * 16, 16)] = zv
            bufb[i, pl.ds(k * 16, 16)] = zv
        return _

    lax.fori_loop(0, CH, zrow, None)
    r0 = sid * ROWS_PER_TILE
    for t in range(ROWS_PER_TILE // CH):
        pltpu.sync_copy(bufa, acc_sh.at[pl.ds(r0 + t * CH, CH)])
    plsc.subcore_barrier()

    pltpu.async_copy(bufb, acc_sh.at[ig.at[0, 0, 1]], semsb, add=True)
    pltpu.async_copy(y_hbm.at[ig.at[0, 0, 0]], bufa, sema)

    def gbody(g, _):
        gpar = lax.rem(g, 2)
        npar = 1 - gpar
        pltpu.make_async_copy(idx_hbm.at[wid, 0], ig.at[0], semi).wait()
        for k in range(GRP):
            b = k % 2
            nb = (k + 1) % 2
            pltpu.make_async_copy(
                y_hbm.at[ig.at[0, 0, 0]], bufs[b], gsems[b]).wait()
            pltpu.async_copy(bufs[b], acc_sh.at[ig.at[gpar, k, 1]], ssems[b],
                             add=True)
            pltpu.make_async_copy(
                bufs[nb], acc_sh.at[ig.at[0, 0, 1]], ssems[nb]).wait()
            if k < GRP - 1:
                nidx = ig.at[gpar, k + 1, 0]
            else:
                nidx = ig.at[npar, 0, 0]
            pltpu.async_copy(y_hbm.at[nidx], bufs[nb], gsems[nb])
        nxt = jnp.minimum(g + 2, NG - 1)
        pltpu.async_copy(idx_hbm.at[wid, nxt], ig.at[gpar], semi)
        return _

    lax.fori_loop(0, NG, gbody, None)
    pltpu.make_async_copy(idx_hbm.at[wid, 0], ig.at[0], semi).wait()
    pltpu.make_async_copy(y_hbm.at[ig.at[0, 0, 0]], bufa, sema).wait()
    pltpu.make_async_copy(bufb, acc_sh.at[ig.at[0, 0, 1]], semsb).wait()
    plsc.subcore_barrier()

    for t in range(ROWS_PER_TILE // CH):
        pltpu.sync_copy(acc_sh.at[pl.ds(r0 + t * CH, CH)], bufa)
        pltpu.sync_copy(bufa, u_hbm.at[pl.ds(cid * NPAD + r0 + t * CH, CH)])


@functools.partial(
    pl.kernel,
    out_type=jax.ShapeDtypeStruct((8192, D), jnp.float32),
    mesh=_mesh,
    compiler_params=_sc_params,
    scratch_types=[
        pltpu.VMEM((2, CH), jnp.int32),
        pltpu.VMEM((CH, D), jnp.float32),
        pltpu.SemaphoreType.DMA,
    ],
)
def _gather_kernel(tab_hbm, idx_hbm, o_hbm, idx_v, buf, sem):
    cid = lax.axis_index("c")
    sid = lax.axis_index("s")
    wid = cid * 16 + sid
    pltpu.sync_copy(idx_hbm.at[wid], idx_v)
    for j in range(2):
        pltpu.async_copy(tab_hbm.at[idx_v.at[j]], buf, sem).wait()
        pltpu.sync_copy(buf, o_hbm.at[pl.ds(wid * 2 * CH + j * CH, CH)])


def _prep_body(degp_ref, x0_ref, y_ref, s_ref, s2_ref):
    deg = jnp.sum(degp_ref[...], axis=0)
    pos = deg > 0
    safe = jnp.where(pos, deg, 1.0)
    s = jnp.where(pos, lax.rsqrt(safe), 0.0)
    s2 = jnp.where(pos, 1.0 / safe, 0.0)
    s_ref[...] = s
    s2_ref[...] = s2
    y_ref[...] = x0_ref[...] * s[:, None]


def _comb_body(u_ref, s2_ref, us_ref, y_ref, usum_ref):
    t = u_ref[:NPAD, :] + u_ref[NPAD:, :]
    usum_ref[...] = us_ref[...] + t
    y_ref[...] = t * s2_ref[...][:, None]


def _final_body(u_ref, s_ref, us_ref, x0_ref, out_ref):
    t = u_ref[:NPAD, :] + u_ref[NPAD:, :]
    usum = us_ref[...] + t
    out_ref[...] = x0_ref[...] + usum * s_ref[...][:, None]


def _dot_body(o_ref, score_ref):
    ou = o_ref[:4096, :]
    oi = o_ref[4096:, :]
    score_ref[...] = jnp.sum(ou * oi, axis=1) * (1.0 / 16.0)


_prep_call = pl.pallas_call(
    _prep_body,
    out_shape=(
        jax.ShapeDtypeStruct((NPAD, D), jnp.float32),
        jax.ShapeDtypeStruct((NPAD,), jnp.float32),
        jax.ShapeDtypeStruct((NPAD,), jnp.float32),
    ),
)

_comb_call = pl.pallas_call(
    _comb_body,
    out_shape=(
        jax.ShapeDtypeStruct((NPAD, D), jnp.float32),
        jax.ShapeDtypeStruct((NPAD, D), jnp.float32),
    ),
)

_final_call = pl.pallas_call(
    _final_body,
    out_shape=jax.ShapeDtypeStruct((NPAD, D), jnp.float32),
)

_dot_call = pl.pallas_call(
    _dot_body,
    out_shape=jax.ShapeDtypeStruct((4096,), jnp.float32),
)


def kernel(edge_index, user_index, item_index, user_emb, item_emb):
    x0 = jnp.concatenate([user_emb, item_emb], axis=0)
    x0p = jnp.pad(x0, ((0, NPAD - N_REAL), (0, 0)))
    src = edge_index[0].astype(jnp.int32)
    dst = edge_index[1].astype(jnp.int32)
    padlen = NT * EPT - src.shape[0]
    pad = (N_REAL + jnp.arange(padlen, dtype=jnp.int32) % (NPAD - N_REAL))
    srcp = jnp.concatenate([src, pad]).reshape(NT, NCH, CH)
    dstp = jnp.concatenate([dst, pad]).reshape(NT, NCH, CH)
    idx5 = jnp.stack([srcp, dstp], axis=2).reshape(NT, NG, GRP, 2, CH)
    bidx = jnp.concatenate(
        [user_index.astype(jnp.int32), item_index.astype(jnp.int32)]
    ).reshape(NT, 2, CH)

    deg_parts = _deg_kernel(idx5)
    y, s, s2 = _prep_call(deg_parts, x0p)
    usum = jnp.zeros((NPAD, D), jnp.float32)
    out_full = None
    for layer in range(3):
        u = _prop_kernel(y, idx5)
        if layer < 2:
            y, usum = _comb_call(u, s2, usum)
        else:
            out_full = _final_call(u, s, usum, x0p)

    o = _gather_kernel(out_full, bidx)
    return _dot_call(o)


# speedup vs baseline: 19.0786x; 1.0023x over previous
import functools

import jax
import jax.numpy as jnp
from jax import lax
from jax.experimental import pallas as pl
from jax.experimental.pallas import tpu as pltpu
from jax.experimental.pallas import tpu_sc as plsc

N_REAL = 10000
NPAD = 10240
D = 128
NT = 32
CH = 128
NCH = 80
GRP = 4
NG = NCH // GRP
EPT = NCH * CH
ROWS_PER_TILE = NPAD // 16

_mesh = plsc.VectorSubcoreMesh(core_axis_name="c", subcore_axis_name="s")
_sc_params = pltpu.CompilerParams(needs_layout_passes=False)


@functools.partial(
    pl.kernel,
    out_type=jax.ShapeDtypeStruct((NT, NPAD), jnp.float32),
    mesh=_mesh,
    compiler_params=_sc_params,
    scratch_types=[
        pltpu.VMEM((NG, GRP, 2, CH), jnp.int32),
        pltpu.VMEM((NPAD,), jnp.float32),
    ],
)
def _deg_kernel(idx_hbm, deg_hbm, idx_v, deg_v):
    cid = lax.axis_index("c")
    sid = lax.axis_index("s")
    wid = cid * 16 + sid
    pltpu.sync_copy(idx_hbm.at[wid], idx_v)

    zv = jnp.zeros((16,), jnp.float32)

    def zbody(i, _):
        deg_v[pl.ds(i * 16, 16)] = zv
        return _

    lax.fori_loop(0, NPAD // 16, zbody, None)

    ones = jnp.ones((16,), jnp.float32)

    def gbody(g, _):
        for k in range(GRP):
            def hbody(h, __):
                idx = idx_v[g, k, 1, pl.ds(h * 16, 16)]
                plsc.addupdate_scatter(deg_v, [idx], ones)
                return __
            lax.fori_loop(0, CH // 16, hbody, None)
        return _

    lax.fori_loop(0, NG, gbody, None)
    pltpu.sync_copy(deg_v, deg_hbm.at[wid])


@functools.partial(
    pl.kernel,
    out_type=jax.ShapeDtypeStruct((2 * NPAD, D), jnp.float32),
    mesh=_mesh,
    compiler_params=_sc_params,
    scratch_types=[
        pltpu.VMEM((2, GRP, 2, CH), jnp.int32),
        pltpu.VMEM((CH, D), jnp.float32),
        pltpu.VMEM((CH, D), jnp.float32),
        pltpu.MemorySpace.VMEM_SHARED((NPAD, D), jnp.float32),
        pltpu.SemaphoreType.DMA,
        pltpu.SemaphoreType.DMA,
        pltpu.SemaphoreType.DMA,
        pltpu.SemaphoreType.DMA,
        pltpu.SemaphoreType.DMA,
    ],
)
def _prop_kernel(y_hbm, idx_hbm, u_hbm, ig, bufa, bufb, acc_sh,
                 sema, semb, semsa, semsb, semi):
    cid = lax.axis_index("c")
    sid = lax.axis_index("s")
    wid = cid * 16 + sid
    bufs = (bufa, bufb)
    gsems = (sema, semb)
    ssems = (semsa, semsb)
    BUF_BYTES = CH * D * 4

    pltpu.sync_copy(idx_hbm.at[wid, 0], ig.at[0])
    pltpu.async_copy(idx_hbm.at[wid, 1], ig.at[1], semi)

    zv = jnp.zeros((16,), jnp.float32)

    def zrow(i, _):
        for k in range(D // 16):
            bufa[i, pl.ds(k * 16, 16)] = zv
            bufb[i, pl.ds(k * 16, 16)] = zv
        return _

    lax.fori_loop(0, CH, zrow, None)
    r0 = sid * ROWS_PER_TILE
    for t in range(ROWS_PER_TILE // CH):
        pltpu.sync_copy(bufa, acc_sh.at[pl.ds(r0 + t * CH, CH)])
    plsc.subcore_barrier()

    pltpu.async_copy(bufb, acc_sh.at[ig.at[0, 0, 1]], semsb, add=True)
    pltpu.async_copy(y_hbm.at[ig.at[0, 0, 0]], bufa, sema)

    def gbody(g, _):
        gpar = lax.rem(g, 2)
        npar = 1 - gpar
        pltpu.make_async_copy(idx_hbm.at[wid, 0], ig.at[0], semi).wait()
        for k in range(GRP):
            b = k % 2
            nb = (k + 1) % 2
            pltpu.make_async_copy(
                y_hbm.at[ig.at[0, 0, 0]], bufs[b], gsems[b]).wait()
            pltpu.async_copy(bufs[b], acc_sh.at[ig.at[gpar, k, 1]], ssems[b],
                             add=True)
            pltpu.make_async_copy(
                bufs[nb], acc_sh.at[ig.at[0, 0, 1]], ssems[nb]).wait()
            if k < GRP - 1:
                nidx = ig.at[gpar, k + 1, 0]
            else:
                nidx = ig.at[npar, 0, 0]
            pltpu.async_copy(y_hbm.at[nidx], bufs[nb], gsems[nb])
        nxt = jnp.minimum(g + 2, NG - 1)
        pltpu.async_copy(idx_hbm.at[wid, nxt], ig.at[gpar], semi)
        return _

    lax.fori_loop(0, NG, gbody, None)
    pltpu.make_async_copy(idx_hbm.at[wid, 0], ig.at[0], semi).wait()
    pltpu.make_async_copy(y_hbm.at[ig.at[0, 0, 0]], bufa, sema).wait()
    pltpu.make_async_copy(bufb, acc_sh.at[ig.at[0, 0, 1]], semsb).wait()
    plsc.subcore_barrier()

    pltpu.sync_copy(acc_sh.at[pl.ds(r0, ROWS_PER_TILE)],
                    u_hbm.at[pl.ds(cid * NPAD + r0, ROWS_PER_TILE)])


@functools.partial(
    pl.kernel,
    out_type=jax.ShapeDtypeStruct((8192, D), jnp.float32),
    mesh=_mesh,
    compiler_params=_sc_params,
    scratch_types=[
        pltpu.VMEM((2, CH), jnp.int32),
        pltpu.VMEM((CH, D), jnp.float32),
        pltpu.SemaphoreType.DMA,
    ],
)
def _gather_kernel(tab_hbm, idx_hbm, o_hbm, idx_v, buf, sem):
    cid = lax.axis_index("c")
    sid = lax.axis_index("s")
    wid = cid * 16 + sid
    pltpu.sync_copy(idx_hbm.at[wid], idx_v)
    for j in range(2):
        pltpu.async_copy(tab_hbm.at[idx_v.at[j]], buf, sem).wait()
        pltpu.sync_copy(buf, o_hbm.at[pl.ds(wid * 2 * CH + j * CH, CH)])


def _prep_body(degp_ref, x0_ref, y_ref, s_ref, s2_ref):
    deg = jnp.sum(degp_ref[...], axis=0)
    pos = deg > 0
    safe = jnp.where(pos, deg, 1.0)
    s = jnp.where(pos, lax.rsqrt(safe), 0.0)
    s2 = jnp.where(pos, 1.0 / safe, 0.0)
    s_ref[...] = s
    s2_ref[...] = s2
    y_ref[...] = x0_ref[...] * s[:, None]


def _comb_body(u_ref, s2_ref, us_ref, y_ref, usum_ref):
    t = u_ref[:NPAD, :] + u_ref[NPAD:, :]
    usum_ref[...] = us_ref[...] + t
    y_ref[...] = t * s2_ref[...][:, None]


def _final_body(u_ref, s_ref, us_ref, x0_ref, out_ref):
    t = u_ref[:NPAD, :] + u_ref[NPAD:, :]
    usum = us_ref[...] + t
    out_ref[...] = x0_ref[...] + usum * s_ref[...][:, None]


def _dot_body(o_ref, score_ref):
    ou = o_ref[:4096, :]
    oi = o_ref[4096:, :]
    score_ref[...] = jnp.sum(ou * oi, axis=1) * (1.0 / 16.0)


_prep_call = pl.pallas_call(
    _prep_body,
    out_shape=(
        jax.ShapeDtypeStruct((NPAD, D), jnp.float32),
        jax.ShapeDtypeStruct((NPAD,), jnp.float32),
        jax.ShapeDtypeStruct((NPAD,), jnp.float32),
    ),
)

_comb_call = pl.pallas_call(
    _comb_body,
    out_shape=(
        jax.ShapeDtypeStruct((NPAD, D), jnp.float32),
        jax.ShapeDtypeStruct((NPAD, D), jnp.float32),
    ),
)

_final_call = pl.pallas_call(
    _final_body,
    out_shape=jax.ShapeDtypeStruct((NPAD, D), jnp.float32),
)

_dot_call = pl.pallas_call(
    _dot_body,
    out_shape=jax.ShapeDtypeStruct((4096,), jnp.float32),
)


def kernel(edge_index, user_index, item_index, user_emb, item_emb):
    x0 = jnp.concatenate([user_emb, item_emb], axis=0)
    x0p = jnp.pad(x0, ((0, NPAD - N_REAL), (0, 0)))
    src = edge_index[0].astype(jnp.int32)
    dst = edge_index[1].astype(jnp.int32)
    padlen = NT * EPT - src.shape[0]
    pad = (N_REAL + jnp.arange(padlen, dtype=jnp.int32) % (NPAD - N_REAL))
    srcp = jnp.concatenate([src, pad]).reshape(NT, NCH, CH)
    dstp = jnp.concatenate([dst, pad]).reshape(NT, NCH, CH)
    idx5 = jnp.stack([srcp, dstp], axis=2).reshape(NT, NG, GRP, 2, CH)
    bidx = jnp.concatenate(
        [user_index.astype(jnp.int32), item_index.astype(jnp.int32)]
    ).reshape(NT, 2, CH)

    deg_parts = _deg_kernel(idx5)
    y, s, s2 = _prep_call(deg_parts, x0p)
    usum = jnp.zeros((NPAD, D), jnp.float32)
    out_full = None
    for layer in range(3):
        u = _prop_kernel(y, idx5)
        if layer < 2:
            y, usum = _comb_call(u, s2, usum)
        else:
            out_full = _final_call(u, s, usum, x0p)

    o = _gather_kernel(out_full, bidx)
    return _dot_call(o)


# speedup vs baseline: 21.3108x; 1.1170x over previous
import functools

import jax
import jax.numpy as jnp
from jax import lax
from jax.experimental import pallas as pl
from jax.experimental.pallas import tpu as pltpu
from jax.experimental.pallas import tpu_sc as plsc

N_REAL = 10000
NPAD = 10240
D = 128
NT = 32
CH = 80
NCH = 128
GRP = 4
NG = NCH // GRP
NBUF = 4
EPT = NCH * CH
CHG = 128
ROWS_PER_TILE = NPAD // 16

assert GRP % NBUF == 0 or NBUF % GRP == 0

_mesh = plsc.VectorSubcoreMesh(core_axis_name="c", subcore_axis_name="s")
_sc_params = pltpu.CompilerParams(needs_layout_passes=False)


@functools.partial(
    pl.kernel,
    out_type=jax.ShapeDtypeStruct((NT, NPAD), jnp.float32),
    mesh=_mesh,
    compiler_params=_sc_params,
    scratch_types=[
        pltpu.VMEM((NG, GRP, 2, CH), jnp.int32),
        pltpu.VMEM((NPAD,), jnp.float32),
    ],
)
def _deg_kernel(idx_hbm, deg_hbm, idx_v, deg_v):
    cid = lax.axis_index("c")
    sid = lax.axis_index("s")
    wid = cid * 16 + sid
    pltpu.sync_copy(idx_hbm.at[wid], idx_v)

    zv = jnp.zeros((16,), jnp.float32)

    def zbody(i, _):
        deg_v[pl.ds(i * 16, 16)] = zv
        return _

    lax.fori_loop(0, NPAD // 16, zbody, None)

    ones = jnp.ones((16,), jnp.float32)

    def gbody(g, _):
        for k in range(GRP):
            def hbody(h, __):
                idx = idx_v[g, k, 1, pl.ds(h * 16, 16)]
                plsc.addupdate_scatter(deg_v, [idx], ones)
                return __
            lax.fori_loop(0, CH // 16, hbody, None)
        return _

    lax.fori_loop(0, NG, gbody, None)
    pltpu.sync_copy(deg_v, deg_hbm.at[wid])


@functools.partial(
    pl.kernel,
    out_type=jax.ShapeDtypeStruct((2 * NPAD, D), jnp.float32),
    mesh=_mesh,
    compiler_params=_sc_params,
    scratch_types=[
        pltpu.VMEM((2, GRP, 2, CH), jnp.int32),
        pltpu.VMEM((NBUF, CH, D), jnp.float32),
        pltpu.MemorySpace.VMEM_SHARED((NPAD, D), jnp.float32),
        pltpu.SemaphoreType.DMA((NBUF,)),
        pltpu.SemaphoreType.DMA((NBUF,)),
        pltpu.SemaphoreType.DMA,
    ],
)
def _prop_kernel(y_hbm, idx_hbm, u_hbm, ig, rows, acc_sh, gsem, ssem, semi):
    cid = lax.axis_index("c")
    sid = lax.axis_index("s")
    wid = cid * 16 + sid

    pltpu.sync_copy(idx_hbm.at[wid, 0], ig.at[0])
    pltpu.async_copy(idx_hbm.at[wid, 1], ig.at[1], semi)

    zv = jnp.zeros((16,), jnp.float32)

    def zrow(i, _):
        for n in range(NBUF):
            for k in range(D // 16):
                rows[n, i, pl.ds(k * 16, 16)] = zv
        return _

    lax.fori_loop(0, CH, zrow, None)
    r0 = sid * ROWS_PER_TILE
    for t in range(ROWS_PER_TILE // CH):
        pltpu.sync_copy(rows.at[0], acc_sh.at[pl.ds(r0 + t * CH, CH)])
    plsc.subcore_barrier()

    pltpu.async_copy(rows.at[NBUF - 1], acc_sh.at[ig.at[0, 0, 1]],
                     ssem.at[NBUF - 1], add=True)
    for i in range(NBUF - 1):
        pltpu.async_copy(y_hbm.at[ig.at[0, i, 0]], rows.at[i], gsem.at[i])

    def gbody(g, _):
        gpar = lax.rem(g, 2)
        npar = 1 - gpar
        pltpu.make_async_copy(idx_hbm.at[wid, 0], ig.at[0], semi).wait()
        for k in range(GRP):
            b = k % NBUF
            tb = (b + NBUF - 1) % NBUF
            pltpu.make_async_copy(
                y_hbm.at[ig.at[0, 0, 0]], rows.at[b], gsem.at[b]).wait()
            pltpu.async_copy(rows.at[b], acc_sh.at[ig.at[gpar, k, 1]],
                             ssem.at[b], add=True)
            pltpu.make_async_copy(
                rows.at[tb], acc_sh.at[ig.at[0, 0, 1]], ssem.at[tb]).wait()
            kg = k + NBUF - 1
            if kg < GRP:
                nidx = ig.at[gpar, kg, 0]
            else:
                nidx = ig.at[npar, kg - GRP, 0]
            pltpu.async_copy(y_hbm.at[nidx], rows.at[tb], gsem.at[tb])
        nxt = jnp.minimum(g + 2, NG - 1)
        pltpu.async_copy(idx_hbm.at[wid, nxt], ig.at[gpar], semi)
        return _

    lax.fori_loop(0, NG, gbody, None)
    pltpu.make_async_copy(idx_hbm.at[wid, 0], ig.at[0], semi).wait()
    pltpu.make_async_copy(rows.at[0], acc_sh.at[ig.at[0, 0, 1]],
                          ssem.at[(NCH - 1) % NBUF]).wait()
    for i in range(NBUF - 1):
        b = (NCH + i) % NBUF
        pltpu.make_async_copy(
            y_hbm.at[ig.at[0, 0, 0]], rows.at[b], gsem.at[b]).wait()
    plsc.subcore_barrier()

    pltpu.sync_copy(acc_sh.at[pl.ds(r0, ROWS_PER_TILE)],
                    u_hbm.at[pl.ds(cid * NPAD + r0, ROWS_PER_TILE)])


@functools.partial(
    pl.kernel,
    out_type=jax.ShapeDtypeStruct((8192, D), jnp.float32),
    mesh=_mesh,
    compiler_params=_sc_params,
    scratch_types=[
        pltpu.VMEM((2, CHG), jnp.int32),
        pltpu.VMEM((CHG, D), jnp.float32),
        pltpu.SemaphoreType.DMA,
    ],
)
def _gather_kernel(tab_hbm, idx_hbm, o_hbm, idx_v, buf, sem):
    cid = lax.axis_index("c")
    sid = lax.axis_index("s")
    wid = cid * 16 + sid
    pltpu.sync_copy(idx_hbm.at[wid], idx_v)
    for j in range(2):
        pltpu.async_copy(tab_hbm.at[idx_v.at[j]], buf, sem).wait()
        pltpu.sync_copy(buf, o_hbm.at[pl.ds(wid * 2 * CHG + j * CHG, CHG)])


def _prep_body(degp_ref, x0_ref, y_ref, s_ref, s2_ref):
    deg = jnp.sum(degp_ref[...], axis=0)
    pos = deg > 0
    safe = jnp.where(pos, deg, 1.0)
    s = jnp.where(pos, lax.rsqrt(safe), 0.0)
    s2 = jnp.where(pos, 1.0 / safe, 0.0)
    s_ref[...] = s
    s2_ref[...] = s2
    y_ref[...] = x0_ref[...] * s[:, None]


def _comb_body(u_ref, s2_ref, us_ref, y_ref, usum_ref):
    t = u_ref[:NPAD, :] + u_ref[NPAD:, :]
    usum_ref[...] = us_ref[...] + t
    y_ref[...] = t * s2_ref[...][:, None]


def _final_body(u_ref, s_ref, us_ref, x0_ref, out_ref):
    t = u_ref[:NPAD, :] + u_ref[NPAD:, :]
    usum = us_ref[...] + t
    out_ref[...] = x0_ref[...] + usum * s_ref[...][:, None]


def _dot_body(o_ref, score_ref):
    ou = o_ref[:4096, :]
    oi = o_ref[4096:, :]
    score_ref[...] = jnp.sum(ou * oi, axis=1) * (1.0 / 16.0)


_prep_call = pl.pallas_call(
    _prep_body,
    out_shape=(
        jax.ShapeDtypeStruct((NPAD, D), jnp.float32),
        jax.ShapeDtypeStruct((NPAD,), jnp.float32),
        jax.ShapeDtypeStruct((NPAD,), jnp.float32),
    ),
)

_comb_call = pl.pallas_call(
    _comb_body,
    out_shape=(
        jax.ShapeDtypeStruct((NPAD, D), jnp.float32),
        jax.ShapeDtypeStruct((NPAD, D), jnp.float32),
    ),
)

_final_call = pl.pallas_call(
    _final_body,
    out_shape=jax.ShapeDtypeStruct((NPAD, D), jnp.float32),
)

_dot_call = pl.pallas_call(
    _dot_body,
    out_shape=jax.ShapeDtypeStruct((4096,), jnp.float32),
)


def kernel(edge_index, user_index, item_index, user_emb, item_emb):
    x0 = jnp.concatenate([user_emb, item_emb], axis=0)
    x0p = jnp.pad(x0, ((0, NPAD - N_REAL), (0, 0)))
    src = edge_index[0].astype(jnp.int32)
    dst = edge_index[1].astype(jnp.int32)
    padlen = NT * EPT - src.shape[0]
    pad = (N_REAL + jnp.arange(padlen, dtype=jnp.int32) % (NPAD - N_REAL))
    srcp = jnp.concatenate([src, pad]).reshape(NT, NCH, CH)
    dstp = jnp.concatenate([dst, pad]).reshape(NT, NCH, CH)
    idx5 = jnp.stack([srcp, dstp], axis=2).reshape(NT, NG, GRP, 2, CH)
    bidx = jnp.concatenate(
        [user_index.astype(jnp.int32), item_index.astype(jnp.int32)]
    ).reshape(NT, 2, CHG)

    deg_parts = _deg_kernel(idx5)
    y, s, s2 = _prep_call(deg_parts, x0p)
    usum = jnp.zeros((NPAD, D), jnp.float32)
    out_full = None
    for layer in range(3):
        u = _prop_kernel(y, idx5)
        if layer < 2:
            y, usum = _comb_call(u, s2, usum)
        else:
            out_full = _final_call(u, s, usum, x0p)

    o = _gather_kernel(out_full, bidx)
    return _dot_call(o)


# speedup vs baseline: 21.6274x; 1.0149x over previous
import functools

import jax
import jax.numpy as jnp
from jax import lax
from jax.experimental import pallas as pl
from jax.experimental.pallas import tpu as pltpu
from jax.experimental.pallas import tpu_sc as plsc

N_REAL = 10000
NPAD = 10240
D = 128
NT = 32
CH = 64
NCH = 160
GRP = 5
NG = NCH // GRP
NBUF = 5
EPT = NCH * CH
CHG = 128
ROWS_PER_TILE = NPAD // 16

assert GRP % NBUF == 0 or NBUF % GRP == 0

_mesh = plsc.VectorSubcoreMesh(core_axis_name="c", subcore_axis_name="s")
_sc_params = pltpu.CompilerParams(needs_layout_passes=False)


@functools.partial(
    pl.kernel,
    out_type=jax.ShapeDtypeStruct((NT, NPAD), jnp.float32),
    mesh=_mesh,
    compiler_params=_sc_params,
    scratch_types=[
        pltpu.VMEM((NG, GRP, 2, CH), jnp.int32),
        pltpu.VMEM((NPAD,), jnp.float32),
    ],
)
def _deg_kernel(idx_hbm, deg_hbm, idx_v, deg_v):
    cid = lax.axis_index("c")
    sid = lax.axis_index("s")
    wid = cid * 16 + sid
    pltpu.sync_copy(idx_hbm.at[wid], idx_v)

    zv = jnp.zeros((16,), jnp.float32)

    def zbody(i, _):
        deg_v[pl.ds(i * 16, 16)] = zv
        return _

    lax.fori_loop(0, NPAD // 16, zbody, None)

    ones = jnp.ones((16,), jnp.float32)

    def gbody(g, _):
        for k in range(GRP):
            def hbody(h, __):
                idx = idx_v[g, k, 1, pl.ds(h * 16, 16)]
                plsc.addupdate_scatter(deg_v, [idx], ones)
                return __
            lax.fori_loop(0, CH // 16, hbody, None)
        return _

    lax.fori_loop(0, NG, gbody, None)
    pltpu.sync_copy(deg_v, deg_hbm.at[wid])


@functools.partial(
    pl.kernel,
    out_type=jax.ShapeDtypeStruct((2 * NPAD, D), jnp.float32),
    mesh=_mesh,
    compiler_params=_sc_params,
    scratch_types=[
        pltpu.VMEM((2, GRP, 2, CH), jnp.int32),
        pltpu.VMEM((NBUF, CH, D), jnp.float32),
        pltpu.MemorySpace.VMEM_SHARED((NPAD, D), jnp.float32),
        pltpu.SemaphoreType.DMA((NBUF,)),
        pltpu.SemaphoreType.DMA((NBUF,)),
        pltpu.SemaphoreType.DMA,
    ],
)
def _prop_kernel(y_hbm, idx_hbm, u_hbm, ig, rows, acc_sh, gsem, ssem, semi):
    cid = lax.axis_index("c")
    sid = lax.axis_index("s")
    wid = cid * 16 + sid

    pltpu.sync_copy(idx_hbm.at[wid, 0], ig.at[0])
    pltpu.async_copy(idx_hbm.at[wid, 1], ig.at[1], semi)

    zv = jnp.zeros((16,), jnp.float32)

    def zrow(i, _):
        for n in range(NBUF):
            for k in range(D // 16):
                rows[n, i, pl.ds(k * 16, 16)] = zv
        return _

    lax.fori_loop(0, CH, zrow, None)
    r0 = sid * ROWS_PER_TILE
    for t in range(ROWS_PER_TILE // CH):
        pltpu.sync_copy(rows.at[0], acc_sh.at[pl.ds(r0 + t * CH, CH)])
    plsc.subcore_barrier()

    pltpu.async_copy(rows.at[NBUF - 1], acc_sh.at[ig.at[0, 0, 1]],
                     ssem.at[NBUF - 1], add=True)
    for i in range(NBUF - 1):
        pltpu.async_copy(y_hbm.at[ig.at[0, i, 0]], rows.at[i], gsem.at[i])

    def gbody(g, _):
        gpar = lax.rem(g, 2)
        npar = 1 - gpar
        pltpu.make_async_copy(idx_hbm.at[wid, 0], ig.at[0], semi).wait()
        for k in range(GRP):
            b = k % NBUF
            tb = (b + NBUF - 1) % NBUF
            pltpu.make_async_copy(
                y_hbm.at[ig.at[0, 0, 0]], rows.at[b], gsem.at[b]).wait()
            pltpu.async_copy(rows.at[b], acc_sh.at[ig.at[gpar, k, 1]],
                             ssem.at[b], add=True)
            pltpu.make_async_copy(
                rows.at[tb], acc_sh.at[ig.at[0, 0, 1]], ssem.at[tb]).wait()
            kg = k + NBUF - 1
            if kg < GRP:
                nidx = ig.at[gpar, kg, 0]
            else:
                nidx = ig.at[npar, kg - GRP, 0]
            pltpu.async_copy(y_hbm.at[nidx], rows.at[tb], gsem.at[tb])
        nxt = jnp.minimum(g + 2, NG - 1)
        pltpu.async_copy(idx_hbm.at[wid, nxt], ig.at[gpar], semi)
        return _

    lax.fori_loop(0, NG, gbody, None)
    pltpu.make_async_copy(idx_hbm.at[wid, 0], ig.at[0], semi).wait()
    pltpu.make_async_copy(rows.at[0], acc_sh.at[ig.at[0, 0, 1]],
                          ssem.at[(NCH - 1) % NBUF]).wait()
    for i in range(NBUF - 1):
        b = (NCH + i) % NBUF
        pltpu.make_async_copy(
            y_hbm.at[ig.at[0, 0, 0]], rows.at[b], gsem.at[b]).wait()
    plsc.subcore_barrier()

    pltpu.sync_copy(acc_sh.at[pl.ds(r0, ROWS_PER_TILE)],
                    u_hbm.at[pl.ds(cid * NPAD + r0, ROWS_PER_TILE)])


@functools.partial(
    pl.kernel,
    out_type=jax.ShapeDtypeStruct((8192, D), jnp.float32),
    mesh=_mesh,
    compiler_params=_sc_params,
    scratch_types=[
        pltpu.VMEM((2, CHG), jnp.int32),
        pltpu.VMEM((CHG, D), jnp.float32),
        pltpu.SemaphoreType.DMA,
    ],
)
def _gather_kernel(tab_hbm, idx_hbm, o_hbm, idx_v, buf, sem):
    cid = lax.axis_index("c")
    sid = lax.axis_index("s")
    wid = cid * 16 + sid
    pltpu.sync_copy(idx_hbm.at[wid], idx_v)
    for j in range(2):
        pltpu.async_copy(tab_hbm.at[idx_v.at[j]], buf, sem).wait()
        pltpu.sync_copy(buf, o_hbm.at[pl.ds(wid * 2 * CHG + j * CHG, CHG)])


def _prep_body(degp_ref, x0_ref, y_ref, s_ref, s2_ref):
    deg = jnp.sum(degp_ref[...], axis=0)
    pos = deg > 0
    safe = jnp.where(pos, deg, 1.0)
    s = jnp.where(pos, lax.rsqrt(safe), 0.0)
    s2 = jnp.where(pos, 1.0 / safe, 0.0)
    s_ref[...] = s
    s2_ref[...] = s2
    y_ref[...] = x0_ref[...] * s[:, None]


def _comb_body(u_ref, s2_ref, us_ref, y_ref, usum_ref):
    t = u_ref[:NPAD, :] + u_ref[NPAD:, :]
    usum_ref[...] = us_ref[...] + t
    y_ref[...] = t * s2_ref[...][:, None]


def _final_body(u_ref, s_ref, us_ref, x0_ref, out_ref):
    t = u_ref[:NPAD, :] + u_ref[NPAD:, :]
    usum = us_ref[...] + t
    out_ref[...] = x0_ref[...] + usum * s_ref[...][:, None]


def _dot_body(o_ref, score_ref):
    ou = o_ref[:4096, :]
    oi = o_ref[4096:, :]
    score_ref[...] = jnp.sum(ou * oi, axis=1) * (1.0 / 16.0)


_prep_call = pl.pallas_call(
    _prep_body,
    out_shape=(
        jax.ShapeDtypeStruct((NPAD, D), jnp.float32),
        jax.ShapeDtypeStruct((NPAD,), jnp.float32),
        jax.ShapeDtypeStruct((NPAD,), jnp.float32),
    ),
)

_comb_call = pl.pallas_call(
    _comb_body,
    out_shape=(
        jax.ShapeDtypeStruct((NPAD, D), jnp.float32),
        jax.ShapeDtypeStruct((NPAD, D), jnp.float32),
    ),
)

_final_call = pl.pallas_call(
    _final_body,
    out_shape=jax.ShapeDtypeStruct((NPAD, D), jnp.float32),
)

_dot_call = pl.pallas_call(
    _dot_body,
    out_shape=jax.ShapeDtypeStruct((4096,), jnp.float32),
)


def kernel(edge_index, user_index, item_index, user_emb, item_emb):
    x0 = jnp.concatenate([user_emb, item_emb], axis=0)
    x0p = jnp.pad(x0, ((0, NPAD - N_REAL), (0, 0)))
    src = edge_index[0].astype(jnp.int32)
    dst = edge_index[1].astype(jnp.int32)
    padlen = NT * EPT - src.shape[0]
    pad = (N_REAL + jnp.arange(padlen, dtype=jnp.int32) % (NPAD - N_REAL))
    srcp = jnp.concatenate([src, pad]).reshape(NT, NCH, CH)
    dstp = jnp.concatenate([dst, pad]).reshape(NT, NCH, CH)
    idx5 = jnp.stack([srcp, dstp], axis=2).reshape(NT, NG, GRP, 2, CH)
    bidx = jnp.concatenate(
        [user_index.astype(jnp.int32), item_index.astype(jnp.int32)]
    ).reshape(NT, 2, CHG)

    deg_parts = _deg_kernel(idx5)
    y, s, s2 = _prep_call(deg_parts, x0p)
    usum = jnp.zeros((NPAD, D), jnp.float32)
    out_full = None
    for layer in range(3):
        u = _prop_kernel(y, idx5)
        if layer < 2:
            y, usum = _comb_call(u, s2, usum)
        else:
            out_full = _final_call(u, s, usum, x0p)

    o = _gather_kernel(out_full, bidx)
    return _dot_call(o)
